# baseline (speedup 1.0000x reference)
import functools
import math

import jax
import jax.numpy as jnp
from jax import lax
from jax.experimental import pallas as pl
from jax.experimental.pallas import tpu as pltpu
from jax.experimental.pallas import tpu_sc as plsc

D_MODEL = 64
SCALE = math.sqrt(D_MODEL)

_info = plsc.get_sparse_core_info()
_NC, _NS, _L = _info.num_cores, _info.num_subcores, _info.num_lanes
_NW = _NC * _NS


def _make_gather(B: int, D: int, chunk: int):
    assert B % (_NW * chunk) == 0
    b_per_w = B // _NW
    n_chunks = b_per_w // chunk
    mesh = plsc.VectorSubcoreMesh(core_axis_name="c", subcore_axis_name="s")

    @functools.partial(
        pl.kernel,
        mesh=mesh,
        out_type=jax.ShapeDtypeStruct((B, D), jnp.float32),
        scratch_types=[
            pltpu.VMEM((chunk,), jnp.int32),
            pltpu.VMEM((chunk, D), jnp.float32),
            pltpu.SemaphoreType.DMA,
        ],
        compiler_params=pltpu.CompilerParams(use_tc_tiling_on_sc=False),
    )
    def gather_scale(table_hbm, idx_hbm, out_hbm, idx_v, rows_v, sem):
        wid = lax.axis_index("s") * _NC + lax.axis_index("c")
        base = wid * b_per_w

        def chunk_body(ci, _):
            off = base + ci * chunk
            pltpu.sync_copy(idx_hbm.at[pl.ds(off, chunk)], idx_v)
            pltpu.async_copy(table_hbm.at[idx_v], rows_v, sem).wait()

            def row_body(r, _):
                for j in range(D // _L):
                    sl = pl.ds(j * _L, _L)
                    rows_v[r, sl] = rows_v[r, sl] * SCALE
                return 0

            lax.fori_loop(0, chunk, row_body, 0)
            pltpu.sync_copy(rows_v, out_hbm.at[pl.ds(off, chunk)])
            return 0

        lax.fori_loop(0, n_chunks, chunk_body, 0)

    return gather_scale


def kernel(x, emb_weight):
    S0, S1 = x.shape
    B = S0 * S1
    idx = jnp.asarray(x, jnp.int32).reshape(B)
    out = _make_gather(B, D_MODEL, 512)(emb_weight, idx)
    return out.reshape(S0, S1, D_MODEL)


# speedup vs baseline: 1.1323x; 1.1323x over previous
import functools
import math

import jax
import jax.numpy as jnp
from jax import lax
from jax.experimental import pallas as pl
from jax.experimental.pallas import tpu as pltpu
from jax.experimental.pallas import tpu_sc as plsc

D_MODEL = 64
SCALE = math.sqrt(D_MODEL)

_info = plsc.get_sparse_core_info()
_NC, _NS, _L = _info.num_cores, _info.num_subcores, _info.num_lanes
_NW = _NC * _NS

_NBUF = 4
_LOOKAHEAD = 2


def _make_gather(B: int, D: int, chunk: int):
    assert B % (_NW * chunk * _NBUF) == 0
    b_per_w = B // _NW
    n_chunks = b_per_w // chunk
    mesh = plsc.VectorSubcoreMesh(core_axis_name="c", subcore_axis_name="s")

    @functools.partial(
        pl.kernel,
        mesh=mesh,
        out_type=jax.ShapeDtypeStruct((B, D), jnp.float32),
        scratch_types=[
            pltpu.VMEM((b_per_w,), jnp.int32),
            [pltpu.VMEM((chunk, D), jnp.float32) for _ in range(_NBUF)],
            [pltpu.SemaphoreType.DMA for _ in range(_NBUF)],
            [pltpu.SemaphoreType.DMA for _ in range(_NBUF)],
        ],
        compiler_params=pltpu.CompilerParams(use_tc_tiling_on_sc=False),
    )
    def gather_scale(table_hbm, idx_hbm, out_hbm, idx_v, bufs, gsems, ssems):
        wid = lax.axis_index("s") * _NC + lax.axis_index("c")
        base = wid * b_per_w
        pltpu.sync_copy(idx_hbm.at[pl.ds(base, b_per_w)], idx_v)

        def issue_gather(ci, b):
            pltpu.async_copy(
                table_hbm.at[idx_v.at[pl.ds(ci * chunk, chunk)]],
                bufs[b], gsems[b])

        def wait_gather(b):
            pltpu.make_async_copy(
                out_hbm.at[pl.ds(0, chunk)], bufs[b], gsems[b]).wait()

        def issue_scatter(ci, b):
            pltpu.async_copy(
                bufs[b], out_hbm.at[pl.ds(base + ci * chunk, chunk)], ssems[b])

        def wait_scatter(b):
            pltpu.make_async_copy(
                bufs[b], out_hbm.at[pl.ds(0, chunk)], ssems[b]).wait()

        for ci in range(_LOOKAHEAD):
            issue_gather(ci, ci % _NBUF)

        def group_body(g, _):
            for b in range(_NBUF):
                ci = g * _NBUF + b
                wait_gather(b)

                def row_body(r, _):
                    for j in range(D // _L):
                        sl = pl.ds(j * _L, _L)
                        bufs[b][r, sl] = bufs[b][r, sl] * SCALE
                    return 0

                lax.fori_loop(0, chunk, row_body, 0)
                issue_scatter(ci, b)
                nci = ci + _LOOKAHEAD
                nb = (b + _LOOKAHEAD) % _NBUF

                @pl.when(nci < n_chunks)
                def _():
                    @pl.when(nci >= _NBUF)
                    def _():
                        wait_scatter(nb)

                    issue_gather(nci, nb)

            return 0

        lax.fori_loop(0, n_chunks // _NBUF, group_body, 0)
        for b in range(_NBUF):
            wait_scatter((n_chunks - _NBUF + b) % _NBUF)

    return gather_scale


def kernel(x, emb_weight):
    S0, S1 = x.shape
    B = S0 * S1
    idx = jnp.asarray(x, jnp.int32).reshape(B)
    out = _make_gather(B, D_MODEL, 400)(emb_weight, idx)
    return out.reshape(S0, S1, D_MODEL)
